# baseline (speedup 1.0000x reference)
import functools

import jax
import jax.numpy as jnp
from jax import lax
from jax.experimental import pallas as pl
from jax.experimental.pallas import tpu as pltpu
from jax.experimental.pallas import tpu_sc as plsc

_NC = 2
_NS = 16
_NW = _NC * _NS


def _sc_segment_sum(table, src_g, dst_g, zeros, n_acc, d_pad, k_chunk, cpp):
    n_chunks = src_g.shape[1]
    n_phases = n_chunks // cpp
    rows_per_sub = n_acc // _NS

    mesh = plsc.VectorSubcoreMesh(
        core_axis_name="c", subcore_axis_name="s",
        num_cores=_NC, num_subcores=_NS)

    @functools.partial(
        pl.kernel,
        mesh=mesh,
        out_type=jax.ShapeDtypeStruct((_NC, n_acc, d_pad), jnp.float32),
        scratch_types=[
            pltpu.VMEM((cpp, k_chunk), jnp.int32),
            pltpu.VMEM((cpp, k_chunk), jnp.int32),
            pltpu.VMEM((k_chunk, d_pad), jnp.float32),
            pltpu.VMEM((k_chunk, d_pad), jnp.float32),
            pltpu.VMEM_SHARED((n_acc, d_pad), jnp.float32),
            pltpu.SemaphoreType.DMA,
            pltpu.SemaphoreType.DMA,
        ],
        compiler_params=pltpu.CompilerParams(use_tc_tiling_on_sc=False),
    )
    def seg_kernel(table_hbm, src_hbm, dst_hbm, zeros_hbm, out_hbm,
                   src_v, dst_v, rows0_v, rows1_v, acc_sh, sem0, sem1):
        c = lax.axis_index("c")
        s = lax.axis_index("s")
        wid = s * _NC + c

        pltpu.sync_copy(zeros_hbm.at[pl.ds(s * rows_per_sub, rows_per_sub)],
                        acc_sh.at[pl.ds(s * rows_per_sub, rows_per_sub)])
        plsc.subcore_barrier()

        bufs = ((rows0_v, sem0), (rows1_v, sem1))

        def phase(p, carry):
            pltpu.sync_copy(src_hbm.at[wid, pl.ds(p * cpp, cpp)], src_v)
            pltpu.sync_copy(dst_hbm.at[wid, pl.ds(p * cpp, cpp)], dst_v)
            for b in range(2):
                rows, sem = bufs[b]
                pltpu.async_copy(table_hbm.at[src_v.at[b]], rows, sem)

            def body(j, carry2):
                for b in range(2):
                    i = j * 2 + b
                    rows, sem = bufs[b]
                    pltpu.make_async_copy(
                        table_hbm.at[src_v.at[i]], rows, sem).wait()
                    pltpu.sync_copy(rows, acc_sh.at[dst_v.at[i]], add=True)
                    nxt = i + 2

                    @pl.when(nxt < cpp)
                    def _start():
                        pltpu.async_copy(
                            table_hbm.at[src_v.at[nxt]], rows, sem)
                return carry2

            lax.fori_loop(0, cpp // 2, body, 0)
            return carry

        lax.fori_loop(0, n_phases, phase, 0)

        plsc.subcore_barrier()
        pltpu.sync_copy(acc_sh.at[pl.ds(s * rows_per_sub, rows_per_sub)],
                        out_hbm.at[c, pl.ds(s * rows_per_sub, rows_per_sub)])

    return seg_kernel(table, src_g, dst_g, zeros)


def _tc_stage1(x, acc1, Wl1, bl1, Wr1, W1, b1, Wl2, Wr2, bl2, blk):
    n = x.shape[0]
    d = x.shape[1]
    cdim = Wl2.shape[0]

    def body(x_ref, acc_ref, wl1_ref, bl1_ref, wr1_ref, w1_ref, b1_ref,
             wl2_ref, wr2_ref, bl2_ref, y2p_ref, z2c_ref):
        acc = acc_ref[0] + acc_ref[1]
        agg_sum = acc[:, :d]
        cnt = jnp.maximum(acc[:, d:d + 1], 1.0)
        agg = agg_sum / cnt
        xb = x_ref[...]

        dot = lambda a, w: lax.dot_general(
            a, w, (((1,), (1,)), ((), ())), preferred_element_type=jnp.float32)

        pre1 = dot(agg, wl1_ref[...]) + bl1_ref[...] + dot(xb, wr1_ref[...])
        nrm1 = jnp.sqrt(jnp.sum(pre1 * pre1, axis=1, keepdims=True))
        h1 = jnp.maximum(pre1 / jnp.maximum(nrm1, 1e-12), 0.0)

        w1 = w1_ref[...]
        h2 = jnp.maximum(dot(xb, w1[:, :d]) + dot(h1, w1[:, d:]) + b1_ref[...],
                         0.0)

        y2 = dot(h2, wl2_ref[...])
        bsz = y2.shape[0]
        y2p_ref[...] = jnp.concatenate(
            [y2, jnp.zeros((bsz, 64 - cdim), jnp.float32)], axis=1)
        z2 = dot(h2, wr2_ref[...]) + bl2_ref[...]
        z2c_ref[...] = jnp.concatenate(
            [z2, cnt, jnp.zeros((bsz, 128 - cdim - 1), jnp.float32)], axis=1)

    d_pad = acc1.shape[-1]
    grid = (n // blk,)
    wspec = lambda shp: pl.BlockSpec(shp, lambda i: (0,) * len(shp))
    return pl.pallas_call(
        body,
        grid=grid,
        in_specs=[
            pl.BlockSpec((blk, d), lambda i: (i, 0)),
            pl.BlockSpec((_NC, blk, d_pad), lambda i: (0, i, 0)),
            wspec(Wl1.shape), wspec(bl1.shape), wspec(Wr1.shape),
            wspec(W1.shape), wspec(b1.shape), wspec(Wl2.shape),
            wspec(Wr2.shape), wspec(bl2.shape),
        ],
        out_specs=[
            pl.BlockSpec((blk, 64), lambda i: (i, 0)),
            pl.BlockSpec((blk, 128), lambda i: (i, 0)),
        ],
        out_shape=[
            jax.ShapeDtypeStruct((n, 64), jnp.float32),
            jax.ShapeDtypeStruct((n, 128), jnp.float32),
        ],
    )(x, acc1, Wl1, bl1, Wr1, W1, b1, Wl2, Wr2, bl2)


def _tc_stage2(acc2, z2c, cdim, blk):
    n = z2c.shape[0]

    def body(acc_ref, z2c_ref, out_ref):
        acc = acc_ref[0] + acc_ref[1]
        agg_sum = acc[:, :cdim]
        z2cb = z2c_ref[...]
        z2 = z2cb[:, :cdim]
        cnt = z2cb[:, cdim:cdim + 1]
        pre = agg_sum / cnt + z2
        nrm = jnp.sqrt(jnp.sum(pre * pre, axis=1, keepdims=True))
        out_ref[...] = pre / jnp.maximum(nrm, 1e-12)

    grid = (n // blk,)
    return pl.pallas_call(
        body,
        grid=grid,
        in_specs=[
            pl.BlockSpec((_NC, blk, 64), lambda i: (0, i, 0)),
            pl.BlockSpec((blk, 128), lambda i: (i, 0)),
        ],
        out_specs=pl.BlockSpec((blk, cdim), lambda i: (i, 0)),
        out_shape=jax.ShapeDtypeStruct((n, cdim), jnp.float32),
    )(acc2, z2c)


def kernel(x, Wl1, bl1, Wr1, W1, b1, Wl2, bl2, Wr2, edge_index):
    n, d = x.shape
    e = edge_index.shape[1]
    cdim = Wl2.shape[0]

    d1_pad = 144
    k_chunk = 112
    cpp = 10
    epw_pad = 90 * k_chunk
    e_pad = _NW * epw_pad
    n_chunks = epw_pad // k_chunk
    n_acc = n + 16

    src_p = jnp.concatenate(
        [edge_index[0], jnp.zeros((e_pad - e,), jnp.int32)])
    dst_p = jnp.concatenate(
        [edge_index[1], jnp.full((e_pad - e,), n, jnp.int32)])
    src_g = src_p.reshape(_NW, n_chunks, k_chunk)
    dst_g = dst_p.reshape(_NW, n_chunks, k_chunk)

    table1 = jnp.concatenate(
        [x, jnp.ones((n, 1), jnp.float32),
         jnp.zeros((n, d1_pad - d - 1), jnp.float32)], axis=1)
    zeros1 = jnp.zeros((n_acc, d1_pad), jnp.float32)

    acc1 = _sc_segment_sum(table1, src_g, dst_g, zeros1, n_acc, d1_pad,
                           k_chunk, cpp)

    bl1r = bl1.reshape(1, -1)
    b1r = b1.reshape(1, -1)
    bl2r = bl2.reshape(1, -1)
    y2p, z2c = _tc_stage1(x, acc1, Wl1, bl1r, Wr1, W1, b1r, Wl2, Wr2, bl2r,
                          blk=2000)

    zeros2 = jnp.zeros((n_acc, 64), jnp.float32)
    acc2 = _sc_segment_sum(y2p, src_g, dst_g, zeros2, n_acc, 64, k_chunk, cpp)

    return _tc_stage2(acc2, z2c, cdim, blk=2000)


# speedup vs baseline: 1.0002x; 1.0002x over previous
import functools

import jax
import jax.numpy as jnp
from jax import lax
from jax.experimental import pallas as pl
from jax.experimental.pallas import tpu as pltpu
from jax.experimental.pallas import tpu_sc as plsc

_NC = 2
_NS = 16
_NW = _NC * _NS


def _sc_segment_sum(table, src_g, dst_g, zeros, n_acc, d_pad, k_chunk, cpp):
    n_chunks = src_g.shape[1]
    n_phases = n_chunks // cpp
    rows_per_sub = n_acc // _NS

    mesh = plsc.VectorSubcoreMesh(
        core_axis_name="c", subcore_axis_name="s",
        num_cores=_NC, num_subcores=_NS)

    @functools.partial(
        pl.kernel,
        mesh=mesh,
        out_type=jax.ShapeDtypeStruct((_NC, n_acc, d_pad), jnp.float32),
        scratch_types=[
            pltpu.VMEM((cpp, k_chunk), jnp.int32),
            pltpu.VMEM((cpp, k_chunk), jnp.int32),
            pltpu.VMEM((k_chunk, d_pad), jnp.float32),
            pltpu.VMEM((k_chunk, d_pad), jnp.float32),
            pltpu.VMEM_SHARED((n_acc, d_pad), jnp.float32),
            pltpu.SemaphoreType.DMA,
            pltpu.SemaphoreType.DMA,
        ],
        compiler_params=pltpu.CompilerParams(use_tc_tiling_on_sc=False),
    )
    def seg_kernel(table_hbm, src_hbm, dst_hbm, zeros_hbm, out_hbm,
                   src_v, dst_v, rows0_v, rows1_v, acc_sh, sem0, sem1):
        c = lax.axis_index("c")
        s = lax.axis_index("s")
        wid = s * _NC + c

        pltpu.sync_copy(zeros_hbm.at[pl.ds(s * rows_per_sub, rows_per_sub)],
                        acc_sh.at[pl.ds(s * rows_per_sub, rows_per_sub)])
        plsc.subcore_barrier()

        bufs = ((rows0_v, sem0), (rows1_v, sem1))

        def phase(p, carry):
            pltpu.sync_copy(src_hbm.at[wid, pl.ds(p * cpp, cpp)], src_v)
            pltpu.sync_copy(dst_hbm.at[wid, pl.ds(p * cpp, cpp)], dst_v)
            for b in range(2):
                rows, sem = bufs[b]
                pltpu.async_copy(table_hbm.at[src_v.at[b]], rows, sem)

            def body(j, carry2):
                for b in range(2):
                    i = j * 2 + b
                    rows, sem = bufs[b]
                    pltpu.make_async_copy(
                        table_hbm.at[src_v.at[i]], rows, sem).wait()
                    pltpu.sync_copy(rows, acc_sh.at[dst_v.at[i]], add=True)
                    nxt = i + 2

                    @pl.when(nxt < cpp)
                    def _start():
                        pltpu.async_copy(
                            table_hbm.at[src_v.at[nxt]], rows, sem)
                return carry2

            lax.fori_loop(0, cpp // 2, body, 0)
            return carry

        lax.fori_loop(0, n_phases, phase, 0)

        plsc.subcore_barrier()
        pltpu.sync_copy(acc_sh.at[pl.ds(s * rows_per_sub, rows_per_sub)],
                        out_hbm.at[c, pl.ds(s * rows_per_sub, rows_per_sub)])

    return seg_kernel(table, src_g, dst_g, zeros)


def _tc_stage1(x, acc1, Wl1, bl1, Wr1, W1, b1, Wl2, Wr2, bl2, blk):
    n = x.shape[0]
    d = x.shape[1]
    cdim = Wl2.shape[0]

    def body(x_ref, acc_ref, wl1_ref, bl1_ref, wr1_ref, w1_ref, b1_ref,
             wl2_ref, wr2_ref, bl2_ref, y2p_ref, z2c_ref):
        acc = acc_ref[0] + acc_ref[1]
        agg_sum = acc[:, :d]
        cnt = jnp.maximum(acc[:, d:d + 1], 1.0)
        agg = agg_sum / cnt
        xb = x_ref[...]

        dot = lambda a, w: lax.dot_general(
            a, w, (((1,), (1,)), ((), ())), preferred_element_type=jnp.float32)

        pre1 = dot(agg, wl1_ref[...]) + bl1_ref[...] + dot(xb, wr1_ref[...])
        nrm1 = jnp.sqrt(jnp.sum(pre1 * pre1, axis=1, keepdims=True))
        h1 = jnp.maximum(pre1 / jnp.maximum(nrm1, 1e-12), 0.0)

        w1 = w1_ref[...]
        h2 = jnp.maximum(dot(xb, w1[:, :d]) + dot(h1, w1[:, d:]) + b1_ref[...],
                         0.0)

        y2 = dot(h2, wl2_ref[...])
        bsz = y2.shape[0]
        y2p_ref[...] = jnp.concatenate(
            [y2, jnp.zeros((bsz, 64 - cdim), jnp.float32)], axis=1)
        z2 = dot(h2, wr2_ref[...]) + bl2_ref[...]
        z2c_ref[...] = jnp.concatenate(
            [z2, cnt, jnp.zeros((bsz, 128 - cdim - 1), jnp.float32)], axis=1)

    d_pad = acc1.shape[-1]
    grid = (n // blk,)
    wspec = lambda shp: pl.BlockSpec(shp, lambda i: (0,) * len(shp))
    return pl.pallas_call(
        body,
        grid=grid,
        in_specs=[
            pl.BlockSpec((blk, d), lambda i: (i, 0)),
            pl.BlockSpec((_NC, blk, d_pad), lambda i: (0, i, 0)),
            wspec(Wl1.shape), wspec(bl1.shape), wspec(Wr1.shape),
            wspec(W1.shape), wspec(b1.shape), wspec(Wl2.shape),
            wspec(Wr2.shape), wspec(bl2.shape),
        ],
        out_specs=[
            pl.BlockSpec((blk, 64), lambda i: (i, 0)),
            pl.BlockSpec((blk, 128), lambda i: (i, 0)),
        ],
        out_shape=[
            jax.ShapeDtypeStruct((n, 64), jnp.float32),
            jax.ShapeDtypeStruct((n, 128), jnp.float32),
        ],
    )(x, acc1, Wl1, bl1, Wr1, W1, b1, Wl2, Wr2, bl2)


def _tc_stage2(acc2, z2c, cdim, blk):
    n = z2c.shape[0]

    def body(acc_ref, z2c_ref, out_ref):
        acc = acc_ref[0] + acc_ref[1]
        agg_sum = acc[:, :cdim]
        z2cb = z2c_ref[...]
        z2 = z2cb[:, :cdim]
        cnt = z2cb[:, cdim:cdim + 1]
        pre = agg_sum / cnt + z2
        nrm = jnp.sqrt(jnp.sum(pre * pre, axis=1, keepdims=True))
        out_ref[...] = pre / jnp.maximum(nrm, 1e-12)

    grid = (n // blk,)
    return pl.pallas_call(
        body,
        grid=grid,
        in_specs=[
            pl.BlockSpec((_NC, blk, 64), lambda i: (0, i, 0)),
            pl.BlockSpec((blk, 128), lambda i: (i, 0)),
        ],
        out_specs=pl.BlockSpec((blk, cdim), lambda i: (i, 0)),
        out_shape=jax.ShapeDtypeStruct((n, cdim), jnp.float32),
    )(acc2, z2c)


def kernel(x, Wl1, bl1, Wr1, W1, b1, Wl2, bl2, Wr2, edge_index):
    n, d = x.shape
    e = edge_index.shape[1]
    cdim = Wl2.shape[0]

    d1_pad = 144
    k_chunk = 112
    cpp = 10
    epw_pad = 90 * k_chunk
    e_pad = _NW * epw_pad
    n_chunks = epw_pad // k_chunk
    n_acc = n + 240

    src_p = jnp.concatenate(
        [edge_index[0], jnp.zeros((e_pad - e,), jnp.int32)])
    dst_p = jnp.concatenate(
        [edge_index[1],
         n + (jnp.arange(e_pad - e, dtype=jnp.int32) % (n_acc - n))])
    src_g = src_p.reshape(_NW, n_chunks, k_chunk)
    dst_g = dst_p.reshape(_NW, n_chunks, k_chunk)

    table1 = jnp.concatenate(
        [x, jnp.ones((n, 1), jnp.float32),
         jnp.zeros((n, d1_pad - d - 1), jnp.float32)], axis=1)
    zeros1 = jnp.zeros((n_acc, d1_pad), jnp.float32)

    acc1 = _sc_segment_sum(table1, src_g, dst_g, zeros1, n_acc, d1_pad,
                           k_chunk, cpp)

    bl1r = bl1.reshape(1, -1)
    b1r = b1.reshape(1, -1)
    bl2r = bl2.reshape(1, -1)
    y2p, z2c = _tc_stage1(x, acc1, Wl1, bl1r, Wr1, W1, b1r, Wl2, Wr2, bl2r,
                          blk=2000)

    zeros2 = jnp.zeros((n_acc, 64), jnp.float32)
    acc2 = _sc_segment_sum(y2p, src_g, dst_g, zeros2, n_acc, 64, k_chunk, cpp)

    return _tc_stage2(acc2, z2c, cdim, blk=2000)


# speedup vs baseline: 1.0232x; 1.0230x over previous
import functools

import jax
import jax.numpy as jnp
from jax import lax
from jax.experimental import pallas as pl
from jax.experimental.pallas import tpu as pltpu
from jax.experimental.pallas import tpu_sc as plsc

_NC = 2
_NS = 16
_NW = _NC * _NS


def _sc_segment_sum(table, src_g, dst_g, zeros, n_acc, d_pad, k_chunk, cpp,
                    nc0, nc1):
    np0, np1 = nc0 // cpp, nc1 // cpp
    rows_per_sub = n_acc // _NS

    mesh = plsc.VectorSubcoreMesh(
        core_axis_name="c", subcore_axis_name="s",
        num_cores=_NC, num_subcores=_NS)

    @functools.partial(
        pl.kernel,
        mesh=mesh,
        out_type=jax.ShapeDtypeStruct((_NC, n_acc, d_pad), jnp.float32),
        scratch_types=[
            pltpu.VMEM((cpp, k_chunk), jnp.int32),
            pltpu.VMEM((cpp, k_chunk), jnp.int32),
            pltpu.VMEM((k_chunk, d_pad), jnp.float32),
            pltpu.VMEM((k_chunk, d_pad), jnp.float32),
            pltpu.VMEM_SHARED((n_acc, d_pad), jnp.float32),
            pltpu.SemaphoreType.DMA,
            pltpu.SemaphoreType.DMA,
        ],
        compiler_params=pltpu.CompilerParams(use_tc_tiling_on_sc=False),
    )
    def seg_kernel(table_hbm, src_hbm, dst_hbm, zeros_hbm, out_hbm,
                   src_v, dst_v, rows0_v, rows1_v, acc_sh, sem0, sem1):
        c = lax.axis_index("c")
        s = lax.axis_index("s")
        chunk0 = c * nc0

        pltpu.sync_copy(zeros_hbm.at[pl.ds(s * rows_per_sub, rows_per_sub)],
                        acc_sh.at[pl.ds(s * rows_per_sub, rows_per_sub)])
        plsc.subcore_barrier()

        bufs = ((rows0_v, sem0), (rows1_v, sem1))

        def phase(p, carry):
            pltpu.sync_copy(
                src_hbm.at[s, pl.ds(chunk0 + p * cpp, cpp)], src_v)
            pltpu.sync_copy(
                dst_hbm.at[s, pl.ds(chunk0 + p * cpp, cpp)], dst_v)
            for b in range(2):
                rows, sem = bufs[b]
                pltpu.async_copy(table_hbm.at[src_v.at[b]], rows, sem)

            def body(j, carry2):
                for b in range(2):
                    i = j * 2 + b
                    rows, sem = bufs[b]
                    pltpu.make_async_copy(
                        table_hbm.at[src_v.at[i]], rows, sem).wait()
                    pltpu.sync_copy(rows, acc_sh.at[dst_v.at[i]], add=True)
                    nxt = i + 2

                    @pl.when(nxt < cpp)
                    def _start():
                        pltpu.async_copy(
                            table_hbm.at[src_v.at[nxt]], rows, sem)
                return carry2

            lax.fori_loop(0, cpp // 2, body, 0)
            return carry

        n_phases = np0 + c * (np1 - np0)
        lax.fori_loop(0, n_phases, phase, 0)

        plsc.subcore_barrier()
        pltpu.sync_copy(acc_sh.at[pl.ds(s * rows_per_sub, rows_per_sub)],
                        out_hbm.at[c, pl.ds(s * rows_per_sub, rows_per_sub)])

    return seg_kernel(table, src_g, dst_g, zeros)


def _tc_stage1(x, acc1, Wl1, bl1, Wr1, W1, b1, Wl2, Wr2, bl2, blk):
    n = x.shape[0]
    d = x.shape[1]
    cdim = Wl2.shape[0]

    def body(x_ref, acc_ref, wl1_ref, bl1_ref, wr1_ref, w1_ref, b1_ref,
             wl2_ref, wr2_ref, bl2_ref, y2p_ref, z2c_ref):
        acc = acc_ref[0] + acc_ref[1]
        agg_sum = acc[:, :d]
        cnt = jnp.maximum(acc[:, d:d + 1], 1.0)
        agg = agg_sum / cnt
        xb = x_ref[...]

        dot = lambda a, w: lax.dot_general(
            a, w, (((1,), (1,)), ((), ())), preferred_element_type=jnp.float32)

        pre1 = dot(agg, wl1_ref[...]) + bl1_ref[...] + dot(xb, wr1_ref[...])
        nrm1 = jnp.sqrt(jnp.sum(pre1 * pre1, axis=1, keepdims=True))
        h1 = jnp.maximum(pre1 / jnp.maximum(nrm1, 1e-12), 0.0)

        w1 = w1_ref[...]
        h2 = jnp.maximum(dot(xb, w1[:, :d]) + dot(h1, w1[:, d:]) + b1_ref[...],
                         0.0)

        y2 = dot(h2, wl2_ref[...])
        bsz = y2.shape[0]
        y2p_ref[...] = jnp.concatenate(
            [y2, jnp.zeros((bsz, 64 - cdim), jnp.float32)], axis=1)
        z2 = dot(h2, wr2_ref[...]) + bl2_ref[...]
        z2c_ref[...] = jnp.concatenate(
            [z2, cnt, jnp.zeros((bsz, 128 - cdim - 1), jnp.float32)], axis=1)

    d_pad = acc1.shape[-1]
    grid = (n // blk,)
    wspec = lambda shp: pl.BlockSpec(shp, lambda i: (0,) * len(shp))
    return pl.pallas_call(
        body,
        grid=grid,
        in_specs=[
            pl.BlockSpec((blk, d), lambda i: (i, 0)),
            pl.BlockSpec((_NC, blk, d_pad), lambda i: (0, i, 0)),
            wspec(Wl1.shape), wspec(bl1.shape), wspec(Wr1.shape),
            wspec(W1.shape), wspec(b1.shape), wspec(Wl2.shape),
            wspec(Wr2.shape), wspec(bl2.shape),
        ],
        out_specs=[
            pl.BlockSpec((blk, 64), lambda i: (i, 0)),
            pl.BlockSpec((blk, 128), lambda i: (i, 0)),
        ],
        out_shape=[
            jax.ShapeDtypeStruct((n, 64), jnp.float32),
            jax.ShapeDtypeStruct((n, 128), jnp.float32),
        ],
    )(x, acc1, Wl1, bl1, Wr1, W1, b1, Wl2, Wr2, bl2)


def _tc_stage2(acc2, z2c, cdim, blk):
    n = z2c.shape[0]

    def body(acc_ref, z2c_ref, out_ref):
        acc = acc_ref[0] + acc_ref[1]
        agg_sum = acc[:, :cdim]
        z2cb = z2c_ref[...]
        z2 = z2cb[:, :cdim]
        cnt = z2cb[:, cdim:cdim + 1]
        pre = agg_sum / cnt + z2
        nrm = jnp.sqrt(jnp.sum(pre * pre, axis=1, keepdims=True))
        out_ref[...] = pre / jnp.maximum(nrm, 1e-12)

    grid = (n // blk,)
    return pl.pallas_call(
        body,
        grid=grid,
        in_specs=[
            pl.BlockSpec((_NC, blk, 64), lambda i: (0, i, 0)),
            pl.BlockSpec((blk, 128), lambda i: (i, 0)),
        ],
        out_specs=pl.BlockSpec((blk, cdim), lambda i: (i, 0)),
        out_shape=jax.ShapeDtypeStruct((n, cdim), jnp.float32),
    )(acc2, z2c)


def kernel(x, Wl1, bl1, Wr1, W1, b1, Wl2, bl2, Wr2, edge_index):
    n, d = x.shape
    e = edge_index.shape[1]
    cdim = Wl2.shape[0]

    d1_pad = 144
    k_chunk = 112
    cpp = 10
    nc0, nc1 = 110, 70
    e0 = _NS * nc0 * k_chunk
    e_pad = _NS * (nc0 + nc1) * k_chunk
    n_acc = n + 240

    src_p = jnp.concatenate(
        [edge_index[0], jnp.zeros((e_pad - e,), jnp.int32)])
    dst_p = jnp.concatenate(
        [edge_index[1],
         n + (jnp.arange(e_pad - e, dtype=jnp.int32) % (n_acc - n))])

    def split_chunks(a):
        a0 = a[:e0].reshape(_NS, nc0, k_chunk)
        a1 = a[e0:].reshape(_NS, nc1, k_chunk)
        return jnp.concatenate([a0, a1], axis=1)

    src_g = split_chunks(src_p)
    dst_g = split_chunks(dst_p)

    table1 = jnp.concatenate(
        [x, jnp.ones((n, 1), jnp.float32),
         jnp.zeros((n, d1_pad - d - 1), jnp.float32)], axis=1)
    zeros1 = jnp.zeros((n_acc, d1_pad), jnp.float32)

    acc1 = _sc_segment_sum(table1, src_g, dst_g, zeros1, n_acc, d1_pad,
                           k_chunk, cpp, nc0, nc1)

    bl1r = bl1.reshape(1, -1)
    b1r = b1.reshape(1, -1)
    bl2r = bl2.reshape(1, -1)
    y2p, z2c = _tc_stage1(x, acc1, Wl1, bl1r, Wr1, W1, b1r, Wl2, Wr2, bl2r,
                          blk=2000)

    zeros2 = jnp.zeros((n_acc, 64), jnp.float32)
    acc2 = _sc_segment_sum(y2p, src_g, dst_g, zeros2, n_acc, 64, k_chunk,
                           cpp, nc0, nc1)

    return _tc_stage2(acc2, z2c, cdim, blk=2000)


# speedup vs baseline: 1.2081x; 1.1806x over previous
import functools

import jax
import jax.numpy as jnp
from jax import lax
from jax.experimental import pallas as pl
from jax.experimental.pallas import tpu as pltpu
from jax.experimental.pallas import tpu_sc as plsc

_NC = 2
_NS = 16
_L = 16


def _sc_segment_sum(table, src2d, dst2d, n_acc, d_pad, k_chunk, cpp,
                    nc0, nc1, with_counts):
    np0, np1 = nc0 // cpp, nc1 // cpp
    rps = n_acc // _NS

    mesh = plsc.VectorSubcoreMesh(
        core_axis_name="c", subcore_axis_name="s",
        num_cores=_NC, num_subcores=_NS)

    out_type = [jax.ShapeDtypeStruct((_NC, n_acc, d_pad), jnp.float32)]
    scratch = [
        pltpu.VMEM((cpp, k_chunk), jnp.int32),
        pltpu.VMEM((cpp, k_chunk), jnp.int32),
        pltpu.VMEM((k_chunk, d_pad), jnp.float32),
        pltpu.VMEM((k_chunk, d_pad), jnp.float32),
        pltpu.VMEM_SHARED((n_acc, d_pad), jnp.float32),
        pltpu.SemaphoreType.DMA,
        pltpu.SemaphoreType.DMA,
    ]
    if with_counts:
        out_type.append(
            jax.ShapeDtypeStruct((_NC, _NS, n_acc), jnp.float32))
        scratch.append(pltpu.VMEM((n_acc,), jnp.float32))

    @functools.partial(
        pl.kernel,
        mesh=mesh,
        out_type=out_type,
        scratch_types=scratch,
        compiler_params=pltpu.CompilerParams(use_tc_tiling_on_sc=False, needs_layout_passes=False),
    )
    def seg_kernel(table_hbm, src_hbm, dst_hbm, out_hbm, *rest):
        if with_counts:
            (cnt_hbm, src_v, dst_v, rows0_v, rows1_v, acc_sh, sem0, sem1,
             cnt_v) = rest
        else:
            src_v, dst_v, rows0_v, rows1_v, acc_sh, sem0, sem1 = rest

        c = lax.axis_index("c")
        s = lax.axis_index("s")
        nc_c = nc0 - c * (nc0 - nc1)
        row_base = c * _NS * nc0 + s * nc_c

        zrow = jnp.zeros((_L,), jnp.float32)

        def zfill(r, carry):
            for jj in range(d_pad // _L):
                rows0_v[r, pl.ds(jj * _L, _L)] = zrow
            return carry

        lax.fori_loop(0, k_chunk, zfill, 0)
        nfull, nrem = rps // k_chunk, rps % k_chunk

        def zcopy(q, carry):
            pltpu.sync_copy(
                rows0_v.at[...],
                acc_sh.at[pl.ds(s * rps + q * k_chunk, k_chunk)])
            return carry

        lax.fori_loop(0, nfull, zcopy, 0)
        if nrem:
            pltpu.sync_copy(
                rows0_v.at[pl.ds(0, nrem)],
                acc_sh.at[pl.ds(s * rps + nfull * k_chunk, nrem)])
        if with_counts:
            def czero(r, carry):
                cnt_v[pl.ds(r * _L, _L)] = zrow
                return carry

            lax.fori_loop(0, n_acc // _L, czero, 0)
        plsc.subcore_barrier()

        bufs = ((rows0_v, sem0), (rows1_v, sem1))
        ones = jnp.full((_L,), 1.0, jnp.float32)

        def phase(p, carry):
            pltpu.sync_copy(
                src_hbm.at[pl.ds(row_base + p * cpp, cpp)], src_v)
            pltpu.sync_copy(
                dst_hbm.at[pl.ds(row_base + p * cpp, cpp)], dst_v)
            for b in range(2):
                rows, sem = bufs[b]
                pltpu.async_copy(table_hbm.at[src_v.at[b]], rows, sem)

            def body(j, carry2):
                for b in range(2):
                    i = j * 2 + b
                    rows, sem = bufs[b]
                    pltpu.make_async_copy(
                        table_hbm.at[src_v.at[i]], rows, sem).wait()
                    pltpu.sync_copy(rows, acc_sh.at[dst_v.at[i]], add=True)
                    nxt = i + 2

                    @pl.when(nxt < cpp)
                    def _start():
                        pltpu.async_copy(
                            table_hbm.at[src_v.at[nxt]], rows, sem)
                    if with_counts:
                        for jj in range(k_chunk // _L):
                            dvec = dst_v[i, pl.ds(jj * _L, _L)]
                            plsc.addupdate_scatter(cnt_v, [dvec], ones)
                return carry2

            lax.fori_loop(0, cpp // 2, body, 0)
            return carry

        n_phases = np0 + c * (np1 - np0)
        lax.fori_loop(0, n_phases, phase, 0)

        plsc.subcore_barrier()
        pltpu.sync_copy(acc_sh.at[pl.ds(s * rps, rps)],
                        out_hbm.at[c, pl.ds(s * rps, rps)])
        if with_counts:
            pltpu.sync_copy(cnt_v, cnt_hbm.at[c, s])

    return seg_kernel(table, src2d, dst2d)


def _tc_stage1(x, acc1, cnt1, Wl1, bl1, Wr1, W1, b1, Wl2, Wr2, bl2, blk):
    n = x.shape[0]
    d = x.shape[1]
    cdim = Wl2.shape[0]

    def body(x_ref, acc_ref, cnt_ref, wl1_ref, bl1_ref, wr1_ref, w1_ref,
             b1_ref, wl2_ref, wr2_ref, bl2_ref, y2p_ref, z2c_ref):
        agg_sum = acc_ref[0] + acc_ref[1]
        cnt = jnp.maximum(
            jnp.sum(cnt_ref[...], axis=1, keepdims=True), 1.0)
        agg = agg_sum / cnt
        xb = x_ref[...]

        dot = lambda a, w: lax.dot_general(
            a, w, (((1,), (1,)), ((), ())), preferred_element_type=jnp.float32)

        pre1 = dot(agg, wl1_ref[...]) + bl1_ref[...] + dot(xb, wr1_ref[...])
        nrm1 = jnp.sqrt(jnp.sum(pre1 * pre1, axis=1, keepdims=True))
        h1 = jnp.maximum(pre1 / jnp.maximum(nrm1, 1e-12), 0.0)

        w1 = w1_ref[...]
        h2 = jnp.maximum(dot(xb, w1[:, :d]) + dot(h1, w1[:, d:]) + b1_ref[...],
                         0.0)

        y2 = dot(h2, wl2_ref[...])
        bsz = y2.shape[0]
        y2p_ref[...] = jnp.concatenate(
            [y2, jnp.zeros((bsz, 64 - cdim), jnp.float32)], axis=1)
        z2 = dot(h2, wr2_ref[...]) + bl2_ref[...]
        z2c_ref[...] = jnp.concatenate(
            [z2, cnt, jnp.zeros((bsz, 128 - cdim - 1), jnp.float32)], axis=1)

    grid = (n // blk,)
    wspec = lambda shp: pl.BlockSpec(shp, lambda i: (0,) * len(shp))
    return pl.pallas_call(
        body,
        grid=grid,
        in_specs=[
            pl.BlockSpec((blk, d), lambda i: (i, 0)),
            pl.BlockSpec((_NC, blk, d), lambda i: (0, i, 0)),
            pl.BlockSpec((blk, _NC * _NS), lambda i: (i, 0)),
            wspec(Wl1.shape), wspec(bl1.shape), wspec(Wr1.shape),
            wspec(W1.shape), wspec(b1.shape), wspec(Wl2.shape),
            wspec(Wr2.shape), wspec(bl2.shape),
        ],
        out_specs=[
            pl.BlockSpec((blk, 64), lambda i: (i, 0)),
            pl.BlockSpec((blk, 128), lambda i: (i, 0)),
        ],
        out_shape=[
            jax.ShapeDtypeStruct((n, 64), jnp.float32),
            jax.ShapeDtypeStruct((n, 128), jnp.float32),
        ],
    )(x, acc1, cnt1, Wl1, bl1, Wr1, W1, b1, Wl2, Wr2, bl2)


def _tc_stage2(acc2, z2c, cdim, blk):
    n = z2c.shape[0]

    def body(acc_ref, z2c_ref, out_ref):
        acc = acc_ref[0] + acc_ref[1]
        agg_sum = acc[:, :cdim]
        z2cb = z2c_ref[...]
        z2 = z2cb[:, :cdim]
        cnt = z2cb[:, cdim:cdim + 1]
        pre = agg_sum / cnt + z2
        nrm = jnp.sqrt(jnp.sum(pre * pre, axis=1, keepdims=True))
        out_ref[...] = pre / jnp.maximum(nrm, 1e-12)

    grid = (n // blk,)
    return pl.pallas_call(
        body,
        grid=grid,
        in_specs=[
            pl.BlockSpec((_NC, blk, 64), lambda i: (0, i, 0)),
            pl.BlockSpec((blk, 128), lambda i: (i, 0)),
        ],
        out_specs=pl.BlockSpec((blk, cdim), lambda i: (i, 0)),
        out_shape=jax.ShapeDtypeStruct((n, cdim), jnp.float32),
    )(acc2, z2c)


def kernel(x, Wl1, bl1, Wr1, W1, b1, Wl2, bl2, Wr2, edge_index):
    n, d = x.shape
    e = edge_index.shape[1]
    cdim = Wl2.shape[0]

    k_chunk = 112
    cpp = 10
    nc0, nc1 = 110, 70
    e_pad = _NS * (nc0 + nc1) * k_chunk
    n_acc = n + 240

    src2d = jnp.concatenate(
        [edge_index[0], jnp.zeros((e_pad - e,), jnp.int32)]
    ).reshape(-1, k_chunk)
    dst2d = jnp.concatenate(
        [edge_index[1],
         n + (jnp.arange(e_pad - e, dtype=jnp.int32) % (n_acc - n))]
    ).reshape(-1, k_chunk)

    acc1, cnt1 = _sc_segment_sum(x, src2d, dst2d, n_acc, d, k_chunk, cpp,
                                 nc0, nc1, with_counts=True)

    bl1r = bl1.reshape(1, -1)
    b1r = b1.reshape(1, -1)
    bl2r = bl2.reshape(1, -1)
    cnt1t = cnt1.reshape(_NC * _NS, n_acc).T
    y2p, z2c = _tc_stage1(x, acc1, cnt1t, Wl1, bl1r, Wr1, W1, b1r, Wl2, Wr2,
                          bl2r, blk=2000)

    acc2, = _sc_segment_sum(y2p, src2d, dst2d, n_acc, 64, k_chunk, cpp,
                            nc0, nc1, with_counts=False)

    return _tc_stage2(acc2, z2c, cdim, blk=2000)


# speedup vs baseline: 1.3195x; 1.0922x over previous
import functools

import jax
import jax.numpy as jnp
from jax import lax
from jax.experimental import pallas as pl
from jax.experimental.pallas import tpu as pltpu
from jax.experimental.pallas import tpu_sc as plsc

_NC = 2
_NS = 16
_L = 16


def _sc_segment_sum(table, src2d, dst2d, n_acc, k_chunk, cpp, with_counts):
    n_tab = table.shape[0]
    dh = table.shape[1] // 2
    spc = src2d.shape[0] // _NS
    n_phases = spc // cpp
    rps = n_acc // _NS
    tps = n_tab // _NS

    mesh = plsc.VectorSubcoreMesh(
        core_axis_name="c", subcore_axis_name="s",
        num_cores=_NC, num_subcores=_NS)

    out_type = [jax.ShapeDtypeStruct((_NC, n_acc, dh), jnp.float32)]
    scratch = [
        pltpu.VMEM((cpp, k_chunk), jnp.int32),
        pltpu.VMEM((cpp, k_chunk), jnp.int32),
        pltpu.VMEM((k_chunk, dh), jnp.float32),
        pltpu.VMEM((k_chunk, dh), jnp.float32),
        pltpu.VMEM_SHARED((n_tab, dh), jnp.float32),
        pltpu.VMEM_SHARED((n_acc, dh), jnp.float32),
        pltpu.SemaphoreType.DMA,
        pltpu.SemaphoreType.DMA,
    ]
    if with_counts:
        out_type.append(jax.ShapeDtypeStruct((_NS, n_acc), jnp.float32))
        scratch.append(pltpu.VMEM((n_acc,), jnp.float32))

    @functools.partial(
        pl.kernel,
        mesh=mesh,
        out_type=out_type,
        scratch_types=scratch,
        compiler_params=pltpu.CompilerParams(
            use_tc_tiling_on_sc=False, needs_layout_passes=False),
    )
    def seg_kernel(table_hbm, src_hbm, dst_hbm, out_hbm, *rest):
        if with_counts:
            (cnt_hbm, src_v, dst_v, rows0_v, rows1_v, tab_sh, acc_sh,
             sem0, sem1, cnt_v) = rest
        else:
            (src_v, dst_v, rows0_v, rows1_v, tab_sh, acc_sh,
             sem0, sem1) = rest

        c = lax.axis_index("c")
        s = lax.axis_index("s")
        row_base = s * spc

        pltpu.sync_copy(
            table_hbm.at[pl.ds(s * tps, tps), pl.ds(c * dh, dh)],
            tab_sh.at[pl.ds(s * tps, tps)])

        zrow = jnp.zeros((_L,), jnp.float32)

        def zfill(r, carry):
            for jj in range(dh // _L):
                rows0_v[r, pl.ds(jj * _L, _L)] = zrow
            return carry

        lax.fori_loop(0, k_chunk, zfill, 0)
        nfull, nrem = rps // k_chunk, rps % k_chunk

        def zcopy(q, carry):
            pltpu.sync_copy(
                rows0_v.at[...],
                acc_sh.at[pl.ds(s * rps + q * k_chunk, k_chunk)])
            return carry

        lax.fori_loop(0, nfull, zcopy, 0)
        if nrem:
            pltpu.sync_copy(
                rows0_v.at[pl.ds(0, nrem)],
                acc_sh.at[pl.ds(s * rps + nfull * k_chunk, nrem)])
        if with_counts:
            def czero(r, carry):
                cnt_v[pl.ds(r * _L, _L)] = zrow
                return carry

            lax.fori_loop(0, n_acc // _L, czero, 0)
        plsc.subcore_barrier()

        bufs = ((rows0_v, sem0), (rows1_v, sem1))
        ones = jnp.full((_L,), 1.0, jnp.float32)

        def phase(p, carry):
            pltpu.sync_copy(
                src_hbm.at[pl.ds(row_base + p * cpp, cpp)], src_v)
            pltpu.sync_copy(
                dst_hbm.at[pl.ds(row_base + p * cpp, cpp)], dst_v)
            for b in range(2):
                rows, sem = bufs[b]
                pltpu.async_copy(tab_sh.at[src_v.at[b]], rows, sem)

            def body(j, carry2):
                for b in range(2):
                    i = j * 2 + b
                    rows, sem = bufs[b]
                    pltpu.make_async_copy(
                        tab_sh.at[src_v.at[i]], rows, sem).wait()
                    pltpu.sync_copy(rows, acc_sh.at[dst_v.at[i]], add=True)
                    nxt = i + 2

                    @pl.when(nxt < cpp)
                    def _start():
                        pltpu.async_copy(
                            tab_sh.at[src_v.at[nxt]], rows, sem)
                    if with_counts:
                        @pl.when(c == 0)
                        def _count():
                            for jj in range(k_chunk // _L):
                                dvec = dst_v[i, pl.ds(jj * _L, _L)]
                                plsc.addupdate_scatter(cnt_v, [dvec], ones)
                return carry2

            lax.fori_loop(0, cpp // 2, body, 0)
            return carry

        lax.fori_loop(0, n_phases, phase, 0)

        plsc.subcore_barrier()
        pltpu.sync_copy(acc_sh.at[pl.ds(s * rps, rps)],
                        out_hbm.at[c, pl.ds(s * rps, rps)])
        if with_counts:
            @pl.when(c == 0)
            def _dump_cnt():
                pltpu.sync_copy(cnt_v, cnt_hbm.at[s])

    return seg_kernel(table, src2d, dst2d)


def _tc_stage1(x, acc1, cnt1, Wl1, bl1, Wr1, W1, b1, Wl2, Wr2, bl2, blk):
    n = x.shape[0]
    d = x.shape[1]
    cdim = Wl2.shape[0]

    def body(x_ref, acc_ref, cnt_ref, wl1_ref, bl1_ref, wr1_ref, w1_ref,
             b1_ref, wl2_ref, wr2_ref, bl2_ref, y2p_ref, z2c_ref):
        agg_sum = jnp.concatenate([acc_ref[0], acc_ref[1]], axis=1)
        cnt = jnp.maximum(
            jnp.sum(cnt_ref[...], axis=1, keepdims=True), 1.0)
        agg = agg_sum / cnt
        xb = x_ref[...]

        dot = lambda a, w: lax.dot_general(
            a, w, (((1,), (1,)), ((), ())), preferred_element_type=jnp.float32)

        pre1 = dot(agg, wl1_ref[...]) + bl1_ref[...] + dot(xb, wr1_ref[...])
        nrm1 = jnp.sqrt(jnp.sum(pre1 * pre1, axis=1, keepdims=True))
        h1 = jnp.maximum(pre1 / jnp.maximum(nrm1, 1e-12), 0.0)

        w1 = w1_ref[...]
        h2 = jnp.maximum(dot(xb, w1[:, :d]) + dot(h1, w1[:, d:]) + b1_ref[...],
                         0.0)

        y2 = dot(h2, wl2_ref[...])
        bsz = y2.shape[0]
        y2p_ref[...] = jnp.concatenate(
            [y2, jnp.zeros((bsz, 64 - cdim), jnp.float32)], axis=1)
        z2 = dot(h2, wr2_ref[...]) + bl2_ref[...]
        z2c_ref[...] = jnp.concatenate(
            [z2, cnt, jnp.zeros((bsz, 128 - cdim - 1), jnp.float32)], axis=1)

    grid = (n // blk,)
    wspec = lambda shp: pl.BlockSpec(shp, lambda i: (0,) * len(shp))
    return pl.pallas_call(
        body,
        grid=grid,
        in_specs=[
            pl.BlockSpec((blk, d), lambda i: (i, 0)),
            pl.BlockSpec((_NC, blk, d // 2), lambda i: (0, i, 0)),
            pl.BlockSpec((blk, _NS), lambda i: (i, 0)),
            wspec(Wl1.shape), wspec(bl1.shape), wspec(Wr1.shape),
            wspec(W1.shape), wspec(b1.shape), wspec(Wl2.shape),
            wspec(Wr2.shape), wspec(bl2.shape),
        ],
        out_specs=[
            pl.BlockSpec((blk, 64), lambda i: (i, 0)),
            pl.BlockSpec((blk, 128), lambda i: (i, 0)),
        ],
        out_shape=[
            jax.ShapeDtypeStruct((n, 64), jnp.float32),
            jax.ShapeDtypeStruct((n, 128), jnp.float32),
        ],
    )(x, acc1, cnt1, Wl1, bl1, Wr1, W1, b1, Wl2, Wr2, bl2)


def _tc_stage2(acc2, z2c, cdim, blk):
    n = z2c.shape[0]

    def body(acc_ref, z2c_ref, out_ref):
        agg = jnp.concatenate([acc_ref[0], acc_ref[1]], axis=1)
        agg_sum = agg[:, :cdim]
        z2cb = z2c_ref[...]
        z2 = z2cb[:, :cdim]
        cnt = z2cb[:, cdim:cdim + 1]
        pre = agg_sum / cnt + z2
        nrm = jnp.sqrt(jnp.sum(pre * pre, axis=1, keepdims=True))
        out_ref[...] = pre / jnp.maximum(nrm, 1e-12)

    grid = (n // blk,)
    return pl.pallas_call(
        body,
        grid=grid,
        in_specs=[
            pl.BlockSpec((_NC, blk, 32), lambda i: (0, i, 0)),
            pl.BlockSpec((blk, 128), lambda i: (i, 0)),
        ],
        out_specs=pl.BlockSpec((blk, cdim), lambda i: (i, 0)),
        out_shape=jax.ShapeDtypeStruct((n, cdim), jnp.float32),
    )(acc2, z2c)


def kernel(x, Wl1, bl1, Wr1, W1, b1, Wl2, bl2, Wr2, edge_index):
    n, d = x.shape
    e = edge_index.shape[1]
    cdim = Wl2.shape[0]

    k_chunk = 112
    cpp = 10
    spc = 180
    e_pad = _NS * spc * k_chunk
    n_acc = n + 240

    src2d = jnp.concatenate(
        [edge_index[0], jnp.zeros((e_pad - e,), jnp.int32)]
    ).reshape(-1, k_chunk)
    dst2d = jnp.concatenate(
        [edge_index[1],
         n + (jnp.arange(e_pad - e, dtype=jnp.int32) % (n_acc - n))]
    ).reshape(-1, k_chunk)

    acc1, cnt1 = _sc_segment_sum(x, src2d, dst2d, n_acc, k_chunk, cpp,
                                 with_counts=True)

    bl1r = bl1.reshape(1, -1)
    b1r = b1.reshape(1, -1)
    bl2r = bl2.reshape(1, -1)
    cnt1t = cnt1.T
    y2p, z2c = _tc_stage1(x, acc1, cnt1t, Wl1, bl1r, Wr1, W1, b1r, Wl2, Wr2,
                          bl2r, blk=2000)

    acc2, = _sc_segment_sum(y2p, src2d, dst2d, n_acc, k_chunk, cpp,
                            with_counts=False)

    return _tc_stage2(acc2, z2c, cdim, blk=2000)


# speedup vs baseline: 1.5142x; 1.1476x over previous
import functools

import jax
import jax.numpy as jnp
from jax import lax
from jax.experimental import pallas as pl
from jax.experimental.pallas import tpu as pltpu
from jax.experimental.pallas import tpu_sc as plsc

_NC = 2
_NS = 16
_L = 16


def _sc_segment_sum(table, src2d, dst2d, n_acc, k_chunk, cpp, with_counts):
    n_tab = table.shape[0]
    dh = table.shape[1] // 2
    spc = src2d.shape[0] // _NS
    n_phases = spc // cpp
    rps = n_acc // _NS
    tps = n_tab // _NS

    mesh = plsc.VectorSubcoreMesh(
        core_axis_name="c", subcore_axis_name="s",
        num_cores=_NC, num_subcores=_NS)

    out_type = [jax.ShapeDtypeStruct((_NC, n_acc, dh), jnp.float32)]
    scratch = [
        pltpu.VMEM((cpp, k_chunk), jnp.int32),
        pltpu.VMEM((cpp, k_chunk), jnp.int32),
        pltpu.VMEM((k_chunk, dh), jnp.float32),
        pltpu.VMEM((k_chunk, dh), jnp.float32),
        pltpu.VMEM((k_chunk, dh), jnp.float32),
        pltpu.VMEM((k_chunk, dh), jnp.float32),
        pltpu.VMEM_SHARED((n_tab, dh), jnp.float32),
        pltpu.VMEM_SHARED((n_acc, dh), jnp.float32),
    ] + [pltpu.SemaphoreType.DMA] * 8
    if with_counts:
        out_type.append(jax.ShapeDtypeStruct((_NS, n_acc), jnp.float32))
        scratch.append(pltpu.VMEM((n_acc,), jnp.float32))

    @functools.partial(
        pl.kernel,
        mesh=mesh,
        out_type=out_type,
        scratch_types=scratch,
        compiler_params=pltpu.CompilerParams(
            use_tc_tiling_on_sc=False, needs_layout_passes=False),
    )
    def seg_kernel(table_hbm, src_hbm, dst_hbm, out_hbm, *rest):
        if with_counts:
            (cnt_hbm, src_v, dst_v, r0, r1, r2, r3, tab_sh, acc_sh,
             g0, g1, g2, g3, s0, s1, s2, s3, cnt_v) = rest
        else:
            (src_v, dst_v, r0, r1, r2, r3, tab_sh, acc_sh,
             g0, g1, g2, g3, s0, s1, s2, s3) = rest
        rows_b = (r0, r1, r2, r3)
        gsem = (g0, g1, g2, g3)
        ssem = (s0, s1, s2, s3)
        rows0_v = r0

        c = lax.axis_index("c")
        s = lax.axis_index("s")
        row_base = s * spc

        pltpu.sync_copy(
            table_hbm.at[pl.ds(s * tps, tps), pl.ds(c * dh, dh)],
            tab_sh.at[pl.ds(s * tps, tps)])

        zrow = jnp.zeros((_L,), jnp.float32)

        def zfill(r, carry):
            for jj in range(dh // _L):
                rows0_v[r, pl.ds(jj * _L, _L)] = zrow
            return carry

        lax.fori_loop(0, k_chunk, zfill, 0)
        nfull, nrem = rps // k_chunk, rps % k_chunk

        def zcopy(q, carry):
            pltpu.sync_copy(
                rows0_v.at[...],
                acc_sh.at[pl.ds(s * rps + q * k_chunk, k_chunk)])
            return carry

        lax.fori_loop(0, nfull, zcopy, 0)
        if nrem:
            pltpu.sync_copy(
                rows0_v.at[pl.ds(0, nrem)],
                acc_sh.at[pl.ds(s * rps + nfull * k_chunk, nrem)])
        if with_counts:
            def czero(r, carry):
                cnt_v[pl.ds(r * _L, _L)] = zrow
                return carry

            lax.fori_loop(0, n_acc // _L, czero, 0)
        plsc.subcore_barrier()

        ones = jnp.full((_L,), 1.0, jnp.float32)

        def phase(p, carry):
            pltpu.sync_copy(
                src_hbm.at[pl.ds(row_base + p * cpp, cpp)], src_v)
            pltpu.sync_copy(
                dst_hbm.at[pl.ds(row_base + p * cpp, cpp)], dst_v)
            for b in range(2):
                pltpu.async_copy(tab_sh.at[src_v.at[b]], rows_b[b], gsem[b])

            def body(j, carry2):
                for b in range(4):
                    i = j * 4 + b
                    pltpu.make_async_copy(
                        tab_sh.at[src_v.at[i]], rows_b[b], gsem[b]).wait()
                    pltpu.async_copy(
                        rows_b[b], acc_sh.at[dst_v.at[i]], ssem[b], add=True)
                    b2 = (b + 2) % 4
                    if b >= 2:
                        pltpu.make_async_copy(
                            rows_b[b2], acc_sh.at[dst_v.at[i]],
                            ssem[b2]).wait()
                    else:
                        @pl.when(j > 0)
                        def _drain():
                            pltpu.make_async_copy(
                                rows_b[b2], acc_sh.at[dst_v.at[i]],
                                ssem[b2]).wait()
                    if b < 2:
                        pltpu.async_copy(
                            tab_sh.at[src_v.at[i + 2]], rows_b[b2], gsem[b2])
                    else:
                        @pl.when(j < cpp // 4 - 1)
                        def _pref():
                            pltpu.async_copy(
                                tab_sh.at[src_v.at[i + 2]], rows_b[b2],
                                gsem[b2])
                    if with_counts:
                        @pl.when(c == 0)
                        def _count():
                            for jj in range(k_chunk // _L):
                                dvec = dst_v[i, pl.ds(jj * _L, _L)]
                                plsc.addupdate_scatter(cnt_v, [dvec], ones)
                return carry2

            lax.fori_loop(0, cpp // 4, body, 0)
            for b in ((cpp - 2) % 4, (cpp - 1) % 4):
                pltpu.make_async_copy(
                    rows_b[b], acc_sh.at[dst_v.at[0]], ssem[b]).wait()
            return carry

        lax.fori_loop(0, n_phases, phase, 0)

        plsc.subcore_barrier()
        pltpu.sync_copy(acc_sh.at[pl.ds(s * rps, rps)],
                        out_hbm.at[c, pl.ds(s * rps, rps)])
        if with_counts:
            @pl.when(c == 0)
            def _dump_cnt():
                pltpu.sync_copy(cnt_v, cnt_hbm.at[s])

    return seg_kernel(table, src2d, dst2d)


def _tc_stage1(x, acc1, cnt1, Wl1, bl1, Wr1, W1, b1, Wl2, Wr2, bl2, blk):
    n = x.shape[0]
    d = x.shape[1]
    cdim = Wl2.shape[0]

    def body(x_ref, acc_ref, cnt_ref, wl1_ref, bl1_ref, wr1_ref, w1_ref,
             b1_ref, wl2_ref, wr2_ref, bl2_ref, y2p_ref, z2c_ref):
        agg_sum = jnp.concatenate([acc_ref[0], acc_ref[1]], axis=1)
        cnt = jnp.maximum(
            jnp.sum(cnt_ref[...], axis=1, keepdims=True), 1.0)
        agg = agg_sum / cnt
        xb = x_ref[...]

        dot = lambda a, w: lax.dot_general(
            a, w, (((1,), (1,)), ((), ())), preferred_element_type=jnp.float32)

        pre1 = dot(agg, wl1_ref[...]) + bl1_ref[...] + dot(xb, wr1_ref[...])
        nrm1 = jnp.sqrt(jnp.sum(pre1 * pre1, axis=1, keepdims=True))
        h1 = jnp.maximum(pre1 / jnp.maximum(nrm1, 1e-12), 0.0)

        w1 = w1_ref[...]
        h2 = jnp.maximum(dot(xb, w1[:, :d]) + dot(h1, w1[:, d:]) + b1_ref[...],
                         0.0)

        y2 = dot(h2, wl2_ref[...])
        bsz = y2.shape[0]
        y2p_ref[...] = jnp.concatenate(
            [y2, jnp.zeros((bsz, 64 - cdim), jnp.float32)], axis=1)
        z2 = dot(h2, wr2_ref[...]) + bl2_ref[...]
        z2c_ref[...] = jnp.concatenate(
            [z2, cnt, jnp.zeros((bsz, 128 - cdim - 1), jnp.float32)], axis=1)

    grid = (n // blk,)
    wspec = lambda shp: pl.BlockSpec(shp, lambda i: (0,) * len(shp))
    return pl.pallas_call(
        body,
        grid=grid,
        in_specs=[
            pl.BlockSpec((blk, d), lambda i: (i, 0)),
            pl.BlockSpec((_NC, blk, d // 2), lambda i: (0, i, 0)),
            pl.BlockSpec((blk, _NS), lambda i: (i, 0)),
            wspec(Wl1.shape), wspec(bl1.shape), wspec(Wr1.shape),
            wspec(W1.shape), wspec(b1.shape), wspec(Wl2.shape),
            wspec(Wr2.shape), wspec(bl2.shape),
        ],
        out_specs=[
            pl.BlockSpec((blk, 64), lambda i: (i, 0)),
            pl.BlockSpec((blk, 128), lambda i: (i, 0)),
        ],
        out_shape=[
            jax.ShapeDtypeStruct((n, 64), jnp.float32),
            jax.ShapeDtypeStruct((n, 128), jnp.float32),
        ],
    )(x, acc1, cnt1, Wl1, bl1, Wr1, W1, b1, Wl2, Wr2, bl2)


def _tc_stage2(acc2, z2c, cdim, blk):
    n = z2c.shape[0]

    def body(acc_ref, z2c_ref, out_ref):
        agg = jnp.concatenate([acc_ref[0], acc_ref[1]], axis=1)
        agg_sum = agg[:, :cdim]
        z2cb = z2c_ref[...]
        z2 = z2cb[:, :cdim]
        cnt = z2cb[:, cdim:cdim + 1]
        pre = agg_sum / cnt + z2
        nrm = jnp.sqrt(jnp.sum(pre * pre, axis=1, keepdims=True))
        out_ref[...] = pre / jnp.maximum(nrm, 1e-12)

    grid = (n // blk,)
    return pl.pallas_call(
        body,
        grid=grid,
        in_specs=[
            pl.BlockSpec((_NC, blk, 32), lambda i: (0, i, 0)),
            pl.BlockSpec((blk, 128), lambda i: (i, 0)),
        ],
        out_specs=pl.BlockSpec((blk, cdim), lambda i: (i, 0)),
        out_shape=jax.ShapeDtypeStruct((n, cdim), jnp.float32),
    )(acc2, z2c)


def kernel(x, Wl1, bl1, Wr1, W1, b1, Wl2, bl2, Wr2, edge_index):
    n, d = x.shape
    e = edge_index.shape[1]
    cdim = Wl2.shape[0]

    k_chunk = 112
    cpp = 12
    spc = 180
    e_pad = _NS * spc * k_chunk
    n_acc = n + 240

    src2d = jnp.concatenate(
        [edge_index[0], jnp.zeros((e_pad - e,), jnp.int32)]
    ).reshape(-1, k_chunk)
    dst2d = jnp.concatenate(
        [edge_index[1],
         n + (jnp.arange(e_pad - e, dtype=jnp.int32) % (n_acc - n))]
    ).reshape(-1, k_chunk)

    acc1, cnt1 = _sc_segment_sum(x, src2d, dst2d, n_acc, k_chunk, cpp,
                                 with_counts=True)

    bl1r = bl1.reshape(1, -1)
    b1r = b1.reshape(1, -1)
    bl2r = bl2.reshape(1, -1)
    cnt1t = cnt1.T
    y2p, z2c = _tc_stage1(x, acc1, cnt1t, Wl1, bl1r, Wr1, W1, b1r, Wl2, Wr2,
                          bl2r, blk=2000)

    acc2, = _sc_segment_sum(y2p, src2d, dst2d, n_acc, k_chunk, cpp,
                            with_counts=False)

    return _tc_stage2(acc2, z2c, cdim, blk=2000)


# speedup vs baseline: 1.5211x; 1.0045x over previous
import functools

import jax
import jax.numpy as jnp
from jax import lax
from jax.experimental import pallas as pl
from jax.experimental.pallas import tpu as pltpu
from jax.experimental.pallas import tpu_sc as plsc

_NC = 2
_NS = 16
_L = 16


def _sc_segment_sum(table, src2d, dst2d, n_acc, k_chunk, cpp, with_counts,
                    col_split=True):
    n_tab = table.shape[0]
    dh = table.shape[1] // 2 if col_split else table.shape[1]
    nw = _NS if col_split else _NS * _NC
    spc = src2d.shape[0] // nw
    n_phases = spc // cpp
    rps = n_acc // _NS
    tps = n_tab // _NS

    mesh = plsc.VectorSubcoreMesh(
        core_axis_name="c", subcore_axis_name="s",
        num_cores=_NC, num_subcores=_NS)

    out_type = [jax.ShapeDtypeStruct((_NC, n_acc, dh), jnp.float32)]
    scratch = [
        pltpu.VMEM((cpp, k_chunk), jnp.int32),
        pltpu.VMEM((cpp, k_chunk), jnp.int32),
        pltpu.VMEM((k_chunk, dh), jnp.float32),
        pltpu.VMEM((k_chunk, dh), jnp.float32),
        pltpu.VMEM((k_chunk, dh), jnp.float32),
        pltpu.VMEM((k_chunk, dh), jnp.float32),
        pltpu.VMEM_SHARED((n_tab, dh), jnp.float32),
        pltpu.VMEM_SHARED((n_acc, dh), jnp.float32),
    ] + [pltpu.SemaphoreType.DMA] * 8
    if with_counts:
        out_type.append(jax.ShapeDtypeStruct((_NS, n_acc), jnp.float32))
        scratch.append(pltpu.VMEM((n_acc,), jnp.float32))

    @functools.partial(
        pl.kernel,
        mesh=mesh,
        out_type=out_type,
        scratch_types=scratch,
        compiler_params=pltpu.CompilerParams(
            use_tc_tiling_on_sc=False, needs_layout_passes=False),
    )
    def seg_kernel(table_hbm, src_hbm, dst_hbm, out_hbm, *rest):
        if with_counts:
            (cnt_hbm, src_v, dst_v, r0, r1, r2, r3, tab_sh, acc_sh,
             g0, g1, g2, g3, s0, s1, s2, s3, cnt_v) = rest
        else:
            (src_v, dst_v, r0, r1, r2, r3, tab_sh, acc_sh,
             g0, g1, g2, g3, s0, s1, s2, s3) = rest
        rows_b = (r0, r1, r2, r3)
        gsem = (g0, g1, g2, g3)
        ssem = (s0, s1, s2, s3)
        rows0_v = r0

        c = lax.axis_index("c")
        s = lax.axis_index("s")
        if col_split:
            row_base = s * spc
            col0 = c * dh
        else:
            row_base = (s * _NC + c) * spc
            col0 = 0

        pltpu.sync_copy(
            table_hbm.at[pl.ds(s * tps, tps), pl.ds(col0, dh)],
            tab_sh.at[pl.ds(s * tps, tps)])

        zrow = jnp.zeros((_L,), jnp.float32)

        def zfill(r, carry):
            for jj in range(dh // _L):
                rows0_v[r, pl.ds(jj * _L, _L)] = zrow
            return carry

        lax.fori_loop(0, k_chunk, zfill, 0)
        nfull, nrem = rps // k_chunk, rps % k_chunk

        def zcopy(q, carry):
            pltpu.sync_copy(
                rows0_v.at[...],
                acc_sh.at[pl.ds(s * rps + q * k_chunk, k_chunk)])
            return carry

        lax.fori_loop(0, nfull, zcopy, 0)
        if nrem:
            pltpu.sync_copy(
                rows0_v.at[pl.ds(0, nrem)],
                acc_sh.at[pl.ds(s * rps + nfull * k_chunk, nrem)])
        if with_counts:
            def czero(r, carry):
                cnt_v[pl.ds(r * _L, _L)] = zrow
                return carry

            lax.fori_loop(0, n_acc // _L, czero, 0)
        plsc.subcore_barrier()

        ones = jnp.full((_L,), 1.0, jnp.float32)

        def phase(p, carry):
            pltpu.sync_copy(
                src_hbm.at[pl.ds(row_base + p * cpp, cpp)], src_v)
            pltpu.sync_copy(
                dst_hbm.at[pl.ds(row_base + p * cpp, cpp)], dst_v)
            for b in range(2):
                pltpu.async_copy(tab_sh.at[src_v.at[b]], rows_b[b], gsem[b])

            def body(j, carry2):
                for b in range(4):
                    i = j * 4 + b
                    pltpu.make_async_copy(
                        tab_sh.at[src_v.at[i]], rows_b[b], gsem[b]).wait()
                    pltpu.async_copy(
                        rows_b[b], acc_sh.at[dst_v.at[i]], ssem[b], add=True)
                    b2 = (b + 2) % 4
                    if b >= 2:
                        pltpu.make_async_copy(
                            rows_b[b2], acc_sh.at[dst_v.at[i]],
                            ssem[b2]).wait()
                    else:
                        @pl.when(j > 0)
                        def _drain():
                            pltpu.make_async_copy(
                                rows_b[b2], acc_sh.at[dst_v.at[i]],
                                ssem[b2]).wait()
                    if b < 2:
                        pltpu.async_copy(
                            tab_sh.at[src_v.at[i + 2]], rows_b[b2], gsem[b2])
                    else:
                        @pl.when(j < cpp // 4 - 1)
                        def _pref():
                            pltpu.async_copy(
                                tab_sh.at[src_v.at[i + 2]], rows_b[b2],
                                gsem[b2])
                    if with_counts:
                        @pl.when(c == 0)
                        def _count():
                            for jj in range(k_chunk // _L):
                                dvec = dst_v[i, pl.ds(jj * _L, _L)]
                                plsc.addupdate_scatter(cnt_v, [dvec], ones)
                return carry2

            lax.fori_loop(0, cpp // 4, body, 0)
            for b in ((cpp - 2) % 4, (cpp - 1) % 4):
                pltpu.make_async_copy(
                    rows_b[b], acc_sh.at[dst_v.at[0]], ssem[b]).wait()
            return carry

        lax.fori_loop(0, n_phases, phase, 0)

        plsc.subcore_barrier()
        pltpu.sync_copy(acc_sh.at[pl.ds(s * rps, rps)],
                        out_hbm.at[c, pl.ds(s * rps, rps)])
        if with_counts:
            @pl.when(c == 0)
            def _dump_cnt():
                pltpu.sync_copy(cnt_v, cnt_hbm.at[s])

    return seg_kernel(table, src2d, dst2d)


def _tc_stage1(x, acc1, cnt1, Wl1, bl1, Wr1, W1, b1, Wl2, Wr2, bl2, blk):
    n = x.shape[0]
    d = x.shape[1]
    cdim = Wl2.shape[0]

    def body(x_ref, acc_ref, cnt_ref, wl1_ref, bl1_ref, wr1_ref, w1_ref,
             b1_ref, wl2_ref, wr2_ref, bl2_ref, y2p_ref, z2c_ref):
        agg_sum = jnp.concatenate([acc_ref[0], acc_ref[1]], axis=1)
        cnt = jnp.maximum(
            jnp.sum(cnt_ref[...], axis=1, keepdims=True), 1.0)
        agg = agg_sum / cnt
        xb = x_ref[...]

        dot = lambda a, w: lax.dot_general(
            a, w, (((1,), (1,)), ((), ())), preferred_element_type=jnp.float32)

        pre1 = dot(agg, wl1_ref[...]) + bl1_ref[...] + dot(xb, wr1_ref[...])
        nrm1 = jnp.sqrt(jnp.sum(pre1 * pre1, axis=1, keepdims=True))
        h1 = jnp.maximum(pre1 / jnp.maximum(nrm1, 1e-12), 0.0)

        w1 = w1_ref[...]
        h2 = jnp.maximum(dot(xb, w1[:, :d]) + dot(h1, w1[:, d:]) + b1_ref[...],
                         0.0)

        y2 = dot(h2, wl2_ref[...])
        bsz = y2.shape[0]
        y2p_ref[...] = jnp.concatenate(
            [y2, jnp.zeros((bsz, 64 - cdim), jnp.float32)], axis=1)
        z2 = dot(h2, wr2_ref[...]) + bl2_ref[...]
        z2c_ref[...] = jnp.concatenate(
            [z2, cnt, jnp.zeros((bsz, 128 - cdim - 1), jnp.float32)], axis=1)

    grid = (n // blk,)
    wspec = lambda shp: pl.BlockSpec(shp, lambda i: (0,) * len(shp))
    return pl.pallas_call(
        body,
        grid=grid,
        in_specs=[
            pl.BlockSpec((blk, d), lambda i: (i, 0)),
            pl.BlockSpec((_NC, blk, d // 2), lambda i: (0, i, 0)),
            pl.BlockSpec((blk, _NS), lambda i: (i, 0)),
            wspec(Wl1.shape), wspec(bl1.shape), wspec(Wr1.shape),
            wspec(W1.shape), wspec(b1.shape), wspec(Wl2.shape),
            wspec(Wr2.shape), wspec(bl2.shape),
        ],
        out_specs=[
            pl.BlockSpec((blk, 64), lambda i: (i, 0)),
            pl.BlockSpec((blk, 128), lambda i: (i, 0)),
        ],
        out_shape=[
            jax.ShapeDtypeStruct((n, 64), jnp.float32),
            jax.ShapeDtypeStruct((n, 128), jnp.float32),
        ],
    )(x, acc1, cnt1, Wl1, bl1, Wr1, W1, b1, Wl2, Wr2, bl2)


def _tc_stage2(acc2, z2c, cdim, blk):
    n = z2c.shape[0]

    def body(acc_ref, z2c_ref, out_ref):
        agg = acc_ref[0] + acc_ref[1]
        agg_sum = agg[:, :cdim]
        z2cb = z2c_ref[...]
        z2 = z2cb[:, :cdim]
        cnt = z2cb[:, cdim:cdim + 1]
        pre = agg_sum / cnt + z2
        nrm = jnp.sqrt(jnp.sum(pre * pre, axis=1, keepdims=True))
        out_ref[...] = pre / jnp.maximum(nrm, 1e-12)

    grid = (n // blk,)
    return pl.pallas_call(
        body,
        grid=grid,
        in_specs=[
            pl.BlockSpec((_NC, blk, 64), lambda i: (0, i, 0)),
            pl.BlockSpec((blk, 128), lambda i: (i, 0)),
        ],
        out_specs=pl.BlockSpec((blk, cdim), lambda i: (i, 0)),
        out_shape=jax.ShapeDtypeStruct((n, cdim), jnp.float32),
    )(acc2, z2c)


def kernel(x, Wl1, bl1, Wr1, W1, b1, Wl2, bl2, Wr2, edge_index):
    n, d = x.shape
    e = edge_index.shape[1]
    cdim = Wl2.shape[0]

    k_chunk = 112
    spc = 184
    e_pad = _NS * spc * k_chunk
    n_acc = n + 240

    src2d = jnp.concatenate(
        [edge_index[0], jnp.zeros((e_pad - e,), jnp.int32)]
    ).reshape(-1, k_chunk)
    dst2d = jnp.concatenate(
        [edge_index[1],
         n + (jnp.arange(e_pad - e, dtype=jnp.int32) % (n_acc - n))]
    ).reshape(-1, k_chunk)

    acc1, cnt1 = _sc_segment_sum(x, src2d, dst2d, n_acc, k_chunk, cpp=8,
                                 with_counts=True, col_split=True)

    bl1r = bl1.reshape(1, -1)
    b1r = b1.reshape(1, -1)
    bl2r = bl2.reshape(1, -1)
    cnt1t = cnt1.T
    y2p, z2c = _tc_stage1(x, acc1, cnt1t, Wl1, bl1r, Wr1, W1, b1r, Wl2, Wr2,
                          bl2r, blk=2000)

    acc2, = _sc_segment_sum(y2p, src2d, dst2d, n_acc, k_chunk, cpp=92,
                            with_counts=False, col_split=False)

    return _tc_stage2(acc2, z2c, cdim, blk=2000)


# speedup vs baseline: 1.5909x; 1.0459x over previous
import functools

import jax
import jax.numpy as jnp
from jax import lax
from jax.experimental import pallas as pl
from jax.experimental.pallas import tpu as pltpu
from jax.experimental.pallas import tpu_sc as plsc

_NC = 2
_NS = 16
_L = 16


def _sc_segment_sum(table, src2d, dst2d, n_acc, k_chunk, cpp, with_counts,
                    col_split=True):
    n_tab = table.shape[0]
    dh = table.shape[1] // 2 if col_split else table.shape[1]
    nw = _NS if col_split else _NS * _NC
    spc = src2d.shape[0] // nw
    n_phases = spc // cpp
    rps = n_acc // _NS
    tps = n_tab // _NS

    mesh = plsc.VectorSubcoreMesh(
        core_axis_name="c", subcore_axis_name="s",
        num_cores=_NC, num_subcores=_NS)

    out_type = [jax.ShapeDtypeStruct((_NC, n_acc, dh), jnp.float32)]
    scratch = [
        pltpu.VMEM((cpp, k_chunk), jnp.int32),
        pltpu.VMEM((cpp, k_chunk), jnp.int32),
        pltpu.VMEM((k_chunk, dh), jnp.float32),
        pltpu.VMEM((k_chunk, dh), jnp.float32),
        pltpu.VMEM((k_chunk, dh), jnp.float32),
        pltpu.VMEM((k_chunk, dh), jnp.float32),
        pltpu.VMEM_SHARED((n_tab, dh), jnp.float32),
        pltpu.VMEM_SHARED((n_acc, dh), jnp.float32),
    ] + [pltpu.SemaphoreType.DMA] * 8
    if with_counts:
        out_type.append(jax.ShapeDtypeStruct((_NS, n_acc), jnp.float32))
        scratch.append(pltpu.VMEM((n_acc,), jnp.float32))

    @functools.partial(
        pl.kernel,
        mesh=mesh,
        out_type=out_type,
        scratch_types=scratch,
        compiler_params=pltpu.CompilerParams(
            use_tc_tiling_on_sc=False, needs_layout_passes=False),
    )
    def seg_kernel(table_hbm, src_hbm, dst_hbm, out_hbm, *rest):
        if with_counts:
            (cnt_hbm, src_v, dst_v, r0, r1, r2, r3, tab_sh, acc_sh,
             g0, g1, g2, g3, s0, s1, s2, s3, cnt_v) = rest
        else:
            (src_v, dst_v, r0, r1, r2, r3, tab_sh, acc_sh,
             g0, g1, g2, g3, s0, s1, s2, s3) = rest
        rows_b = (r0, r1, r2, r3)
        gsem = (g0, g1, g2, g3)
        ssem = (s0, s1, s2, s3)
        rows0_v = r0

        c = lax.axis_index("c")
        s = lax.axis_index("s")
        if col_split:
            row_base = s * spc
            col0 = c * dh
        else:
            row_base = (s * _NC + c) * spc
            col0 = 0

        pltpu.sync_copy(
            table_hbm.at[pl.ds(s * tps, tps), pl.ds(col0, dh)],
            tab_sh.at[pl.ds(s * tps, tps)])

        zrow = jnp.zeros((_L,), jnp.float32)

        def zfill(r, carry):
            for jj in range(dh // _L):
                rows0_v[r, pl.ds(jj * _L, _L)] = zrow
            return carry

        lax.fori_loop(0, k_chunk, zfill, 0)
        nfull, nrem = rps // k_chunk, rps % k_chunk

        def zcopy(q, carry):
            pltpu.sync_copy(
                rows0_v.at[...],
                acc_sh.at[pl.ds(s * rps + q * k_chunk, k_chunk)])
            return carry

        lax.fori_loop(0, nfull, zcopy, 0)
        if nrem:
            pltpu.sync_copy(
                rows0_v.at[pl.ds(0, nrem)],
                acc_sh.at[pl.ds(s * rps + nfull * k_chunk, nrem)])
        if with_counts:
            def czero(r, carry):
                cnt_v[pl.ds(r * _L, _L)] = zrow
                return carry

            lax.fori_loop(0, n_acc // _L, czero, 0)
        plsc.subcore_barrier()

        ones = jnp.full((_L,), 1.0, jnp.float32)

        def phase(p, carry):
            pltpu.sync_copy(
                src_hbm.at[pl.ds(row_base + p * cpp, cpp)], src_v)
            pltpu.sync_copy(
                dst_hbm.at[pl.ds(row_base + p * cpp, cpp)], dst_v)
            for b in range(2):
                pltpu.async_copy(tab_sh.at[src_v.at[b]], rows_b[b], gsem[b])

            def body(j, carry2):
                for b in range(4):
                    i = j * 4 + b
                    pltpu.make_async_copy(
                        tab_sh.at[src_v.at[i]], rows_b[b], gsem[b]).wait()
                    pltpu.async_copy(
                        rows_b[b], acc_sh.at[dst_v.at[i]], ssem[b], add=True)
                    b2 = (b + 2) % 4
                    if b >= 2:
                        pltpu.make_async_copy(
                            rows_b[b2], acc_sh.at[dst_v.at[i]],
                            ssem[b2]).wait()
                    else:
                        @pl.when(j > 0)
                        def _drain():
                            pltpu.make_async_copy(
                                rows_b[b2], acc_sh.at[dst_v.at[i]],
                                ssem[b2]).wait()
                    if b < 2:
                        pltpu.async_copy(
                            tab_sh.at[src_v.at[i + 2]], rows_b[b2], gsem[b2])
                    else:
                        @pl.when(j < cpp // 4 - 1)
                        def _pref():
                            pltpu.async_copy(
                                tab_sh.at[src_v.at[i + 2]], rows_b[b2],
                                gsem[b2])
                    if with_counts:
                        @pl.when(c == 0)
                        def _count():
                            for jj in range(k_chunk // _L):
                                dvec = dst_v[i, pl.ds(jj * _L, _L)]
                                plsc.addupdate_scatter(cnt_v, [dvec], ones)
                return carry2

            lax.fori_loop(0, cpp // 4, body, 0)
            for b in ((cpp - 2) % 4, (cpp - 1) % 4):
                pltpu.make_async_copy(
                    rows_b[b], acc_sh.at[dst_v.at[0]], ssem[b]).wait()
            return carry

        lax.fori_loop(0, n_phases, phase, 0)

        plsc.subcore_barrier()
        pltpu.sync_copy(acc_sh.at[pl.ds(s * rps, rps)],
                        out_hbm.at[c, pl.ds(s * rps, rps)])
        if with_counts:
            @pl.when(c == 0)
            def _dump_cnt():
                pltpu.sync_copy(cnt_v, cnt_hbm.at[s])

    return seg_kernel(table, src2d, dst2d)


def _tc_stage1(x, acc1, cnt1, Wl1, bl1, Wr1, W1, b1, Wl2, Wr2, bl2, blk):
    n = x.shape[0]
    d = x.shape[1]
    cdim = Wl2.shape[0]

    def body(x_ref, acc_ref, cnt_ref, wl1_ref, bl1_ref, wr1_ref, w1_ref,
             b1_ref, wl2_ref, wr2_ref, bl2_ref, y2p_ref, z2c_ref):
        agg_sum = jnp.concatenate([acc_ref[0], acc_ref[1]], axis=1)
        cnt = jnp.maximum(
            jnp.sum(cnt_ref[...], axis=1, keepdims=True), 1.0)
        agg = agg_sum / cnt
        xb = x_ref[...]

        dot = lambda a, w: lax.dot_general(
            a, w, (((1,), (1,)), ((), ())), preferred_element_type=jnp.float32)

        pre1 = dot(agg, wl1_ref[...]) + bl1_ref[...] + dot(xb, wr1_ref[...])
        nrm1 = jnp.sqrt(jnp.sum(pre1 * pre1, axis=1, keepdims=True))
        h1 = jnp.maximum(pre1 / jnp.maximum(nrm1, 1e-12), 0.0)

        w1 = w1_ref[...]
        h2 = jnp.maximum(dot(xb, w1[:, :d]) + dot(h1, w1[:, d:]) + b1_ref[...],
                         0.0)

        y2 = dot(h2, wl2_ref[...])
        bsz = y2.shape[0]
        y2p_ref[...] = jnp.concatenate(
            [y2, jnp.zeros((bsz, 64 - cdim), jnp.float32)], axis=1)
        z2 = dot(h2, wr2_ref[...]) + bl2_ref[...]
        z2c_ref[...] = jnp.concatenate(
            [z2, cnt, jnp.zeros((bsz, 128 - cdim - 1), jnp.float32)], axis=1)

    grid = (n // blk,)
    wspec = lambda shp: pl.BlockSpec(shp, lambda i: (0,) * len(shp))
    return pl.pallas_call(
        body,
        grid=grid,
        in_specs=[
            pl.BlockSpec((blk, d), lambda i: (i, 0)),
            pl.BlockSpec((_NC, blk, d // 2), lambda i: (0, i, 0)),
            pl.BlockSpec((blk, _NS), lambda i: (i, 0)),
            wspec(Wl1.shape), wspec(bl1.shape), wspec(Wr1.shape),
            wspec(W1.shape), wspec(b1.shape), wspec(Wl2.shape),
            wspec(Wr2.shape), wspec(bl2.shape),
        ],
        out_specs=[
            pl.BlockSpec((blk, 64), lambda i: (i, 0)),
            pl.BlockSpec((blk, 128), lambda i: (i, 0)),
        ],
        out_shape=[
            jax.ShapeDtypeStruct((n, 64), jnp.float32),
            jax.ShapeDtypeStruct((n, 128), jnp.float32),
        ],
    )(x, acc1, cnt1, Wl1, bl1, Wr1, W1, b1, Wl2, Wr2, bl2)


def _tc_stage2(acc2, z2c, cdim, blk):
    n = z2c.shape[0]

    def body(acc_ref, z2c_ref, out_ref):
        agg = acc_ref[0] + acc_ref[1]
        agg_sum = agg[:, :cdim]
        z2cb = z2c_ref[...]
        z2 = z2cb[:, :cdim]
        cnt = z2cb[:, cdim:cdim + 1]
        pre = agg_sum / cnt + z2
        nrm = jnp.sqrt(jnp.sum(pre * pre, axis=1, keepdims=True))
        out_ref[...] = pre / jnp.maximum(nrm, 1e-12)

    grid = (n // blk,)
    return pl.pallas_call(
        body,
        grid=grid,
        in_specs=[
            pl.BlockSpec((_NC, blk, 64), lambda i: (0, i, 0)),
            pl.BlockSpec((blk, 128), lambda i: (i, 0)),
        ],
        out_specs=pl.BlockSpec((blk, cdim), lambda i: (i, 0)),
        out_shape=jax.ShapeDtypeStruct((n, cdim), jnp.float32),
    )(acc2, z2c)


def kernel(x, Wl1, bl1, Wr1, W1, b1, Wl2, bl2, Wr2, edge_index):
    n, d = x.shape
    e = edge_index.shape[1]
    cdim = Wl2.shape[0]

    k_chunk = 112
    e1 = _NS * 180 * k_chunk
    e2 = _NS * _NC * 92 * k_chunk
    n_acc = n + 240

    def pad_edges(e_pad):
        src = jnp.concatenate(
            [edge_index[0], jnp.zeros((e_pad - e,), jnp.int32)]
        ).reshape(-1, k_chunk)
        dst = jnp.concatenate(
            [edge_index[1],
             n + (jnp.arange(e_pad - e, dtype=jnp.int32) % (n_acc - n))]
        ).reshape(-1, k_chunk)
        return src, dst

    src2d, dst2d = pad_edges(e1)
    src2db, dst2db = pad_edges(e2)

    acc1, cnt1 = _sc_segment_sum(x, src2d, dst2d, n_acc, k_chunk, cpp=12,
                                 with_counts=True, col_split=True)

    bl1r = bl1.reshape(1, -1)
    b1r = b1.reshape(1, -1)
    bl2r = bl2.reshape(1, -1)
    cnt1t = cnt1.T
    y2p, z2c = _tc_stage1(x, acc1, cnt1t, Wl1, bl1r, Wr1, W1, b1r, Wl2, Wr2,
                          bl2r, blk=2000)

    acc2, = _sc_segment_sum(y2p, src2db, dst2db, n_acc, k_chunk, cpp=92,
                            with_counts=False, col_split=False)

    return _tc_stage2(acc2, z2c, cdim, blk=2000)


# speedup vs baseline: 1.6652x; 1.0467x over previous
import functools

import jax
import jax.numpy as jnp
from jax import lax
from jax.experimental import pallas as pl
from jax.experimental.pallas import tpu as pltpu
from jax.experimental.pallas import tpu_sc as plsc

_NC = 2
_NS = 16
_L = 16


def _sc_segment_sum(table, src2d, dst2d, n_acc, k_chunk, cpp, with_counts,
                    col_split=True):
    n_tab = table.shape[0]
    dh = table.shape[1] // 2 if col_split else table.shape[1]
    nw = _NS if col_split else _NS * _NC
    spc = src2d.shape[0] // nw
    n_phases = spc // cpp
    rps = n_acc // _NS
    tps = n_tab // _NS

    mesh = plsc.VectorSubcoreMesh(
        core_axis_name="c", subcore_axis_name="s",
        num_cores=_NC, num_subcores=_NS)

    out_type = [jax.ShapeDtypeStruct((_NC, n_acc, dh), jnp.float32)]
    scratch = [
        pltpu.VMEM((cpp, k_chunk), jnp.int32),
        pltpu.VMEM((cpp, k_chunk), jnp.int32),
        pltpu.VMEM((k_chunk, dh), jnp.float32),
        pltpu.VMEM((k_chunk, dh), jnp.float32),
        pltpu.VMEM((k_chunk, dh), jnp.float32),
        pltpu.VMEM((k_chunk, dh), jnp.float32),
        pltpu.VMEM_SHARED((n_tab, dh), jnp.float32),
        pltpu.VMEM_SHARED((n_acc, dh), jnp.float32),
    ] + [pltpu.SemaphoreType.DMA] * 8
    if with_counts:
        out_type.append(jax.ShapeDtypeStruct((_NS, n_acc), jnp.float32))
        scratch.append(pltpu.VMEM((n_acc,), jnp.float32))

    @functools.partial(
        pl.kernel,
        mesh=mesh,
        out_type=out_type,
        scratch_types=scratch,
        compiler_params=pltpu.CompilerParams(
            use_tc_tiling_on_sc=False, needs_layout_passes=False),
    )
    def seg_kernel(table_hbm, src_hbm, dst_hbm, out_hbm, *rest):
        if with_counts:
            (cnt_hbm, src_v, dst_v, r0, r1, r2, r3, tab_sh, acc_sh,
             g0, g1, g2, g3, s0, s1, s2, s3, cnt_v) = rest
        else:
            (src_v, dst_v, r0, r1, r2, r3, tab_sh, acc_sh,
             g0, g1, g2, g3, s0, s1, s2, s3) = rest
        rows_b = (r0, r1, r2, r3)
        gsem = (g0, g1, g2, g3)
        ssem = (s0, s1, s2, s3)
        rows0_v = r0

        c = lax.axis_index("c")
        s = lax.axis_index("s")
        if col_split:
            row_base = s * spc
            col0 = c * dh
        else:
            row_base = (s * _NC + c) * spc
            col0 = 0

        pltpu.sync_copy(
            table_hbm.at[pl.ds(s * tps, tps), pl.ds(col0, dh)],
            tab_sh.at[pl.ds(s * tps, tps)])

        zrow = jnp.zeros((_L,), jnp.float32)

        def zfill(r, carry):
            for jj in range(dh // _L):
                rows0_v[r, pl.ds(jj * _L, _L)] = zrow
            return carry

        lax.fori_loop(0, k_chunk, zfill, 0)
        nfull, nrem = rps // k_chunk, rps % k_chunk

        def zcopy(q, carry):
            pltpu.sync_copy(
                rows0_v.at[...],
                acc_sh.at[pl.ds(s * rps + q * k_chunk, k_chunk)])
            return carry

        lax.fori_loop(0, nfull, zcopy, 0)
        if nrem:
            pltpu.sync_copy(
                rows0_v.at[pl.ds(0, nrem)],
                acc_sh.at[pl.ds(s * rps + nfull * k_chunk, nrem)])
        if with_counts:
            def czero(r, carry):
                cnt_v[pl.ds(r * _L, _L)] = zrow
                return carry

            lax.fori_loop(0, n_acc // _L, czero, 0)
        plsc.subcore_barrier()

        ones = jnp.full((_L,), 1.0, jnp.float32)

        def phase(p, carry):
            pltpu.sync_copy(
                src_hbm.at[pl.ds(row_base + p * cpp, cpp)], src_v)
            pltpu.sync_copy(
                dst_hbm.at[pl.ds(row_base + p * cpp, cpp)], dst_v)
            for b in range(2):
                pltpu.async_copy(tab_sh.at[src_v.at[b]], rows_b[b], gsem[b])

            def body(j, carry2):
                for b in range(4):
                    i = j * 4 + b
                    pltpu.make_async_copy(
                        tab_sh.at[src_v.at[i]], rows_b[b], gsem[b]).wait()
                    pltpu.async_copy(
                        rows_b[b], acc_sh.at[dst_v.at[i]], ssem[b], add=True)
                    b2 = (b + 2) % 4
                    if b >= 2:
                        pltpu.make_async_copy(
                            rows_b[b2], acc_sh.at[dst_v.at[i]],
                            ssem[b2]).wait()
                    else:
                        @pl.when(j > 0)
                        def _drain():
                            pltpu.make_async_copy(
                                rows_b[b2], acc_sh.at[dst_v.at[i]],
                                ssem[b2]).wait()
                    if b < 2:
                        pltpu.async_copy(
                            tab_sh.at[src_v.at[i + 2]], rows_b[b2], gsem[b2])
                    else:
                        @pl.when(j < cpp // 4 - 1)
                        def _pref():
                            pltpu.async_copy(
                                tab_sh.at[src_v.at[i + 2]], rows_b[b2],
                                gsem[b2])
                    if with_counts:
                        @pl.when(c == 0)
                        def _count():
                            for jj in range(k_chunk // _L):
                                dvec = dst_v[i, pl.ds(jj * _L, _L)]
                                plsc.addupdate_scatter(cnt_v, [dvec], ones)
                return carry2

            lax.fori_loop(0, cpp // 4, body, 0)
            for b in ((cpp - 2) % 4, (cpp - 1) % 4):
                pltpu.make_async_copy(
                    rows_b[b], acc_sh.at[dst_v.at[0]], ssem[b]).wait()
            return carry

        lax.fori_loop(0, n_phases, phase, 0)

        plsc.subcore_barrier()
        pltpu.sync_copy(acc_sh.at[pl.ds(s * rps, rps)],
                        out_hbm.at[c, pl.ds(s * rps, rps)])
        if with_counts:
            @pl.when(c == 0)
            def _dump_cnt():
                pltpu.sync_copy(cnt_v, cnt_hbm.at[s])

    return seg_kernel(table, src2d, dst2d)


def _tc_stage1(x, acc1, cnt1, Wl1, bl1, Wr1, W1, b1, Wl2, Wr2, bl2, blk):
    n = x.shape[0]
    d = x.shape[1]
    cdim = Wl2.shape[0]

    def body(x_ref, acc_ref, cnt_ref, wl1_ref, bl1_ref, wr1_ref, w1_ref,
             b1_ref, wl2_ref, wr2_ref, bl2_ref, y2p_ref, z2c_ref):
        agg_sum = jnp.concatenate([acc_ref[0], acc_ref[1]], axis=1)
        cnt = jnp.maximum(
            jnp.sum(cnt_ref[...], axis=1, keepdims=True), 1.0)
        agg = agg_sum / cnt
        xb = x_ref[...]

        dot = lambda a, w: lax.dot_general(
            a, w, (((1,), (1,)), ((), ())), preferred_element_type=jnp.float32)

        pre1 = dot(agg, wl1_ref[...]) + bl1_ref[...] + dot(xb, wr1_ref[...])
        nrm1 = jnp.sqrt(jnp.sum(pre1 * pre1, axis=1, keepdims=True))
        h1 = jnp.maximum(pre1 / jnp.maximum(nrm1, 1e-12), 0.0)

        w1 = w1_ref[...]
        h2 = jnp.maximum(dot(xb, w1[:, :d]) + dot(h1, w1[:, d:]) + b1_ref[...],
                         0.0)

        y2 = dot(h2, wl2_ref[...])
        bsz = y2.shape[0]
        y2p_ref[...] = jnp.concatenate(
            [y2, jnp.zeros((bsz, 64 - cdim), jnp.float32)], axis=1)
        z2 = dot(h2, wr2_ref[...]) + bl2_ref[...]
        z2c_ref[...] = jnp.concatenate(
            [z2, cnt, jnp.zeros((bsz, 128 - cdim - 1), jnp.float32)], axis=1)

    grid = (n // blk,)
    wspec = lambda shp: pl.BlockSpec(shp, lambda i: (0,) * len(shp))
    return pl.pallas_call(
        body,
        grid=grid,
        in_specs=[
            pl.BlockSpec((blk, d), lambda i: (i, 0)),
            pl.BlockSpec((_NC, blk, d // 2), lambda i: (0, i, 0)),
            pl.BlockSpec((blk, _NS), lambda i: (i, 0)),
            wspec(Wl1.shape), wspec(bl1.shape), wspec(Wr1.shape),
            wspec(W1.shape), wspec(b1.shape), wspec(Wl2.shape),
            wspec(Wr2.shape), wspec(bl2.shape),
        ],
        out_specs=[
            pl.BlockSpec((blk, 64), lambda i: (i, 0)),
            pl.BlockSpec((blk, 128), lambda i: (i, 0)),
        ],
        out_shape=[
            jax.ShapeDtypeStruct((n, 64), jnp.float32),
            jax.ShapeDtypeStruct((n, 128), jnp.float32),
        ],
    )(x, acc1, cnt1, Wl1, bl1, Wr1, W1, b1, Wl2, Wr2, bl2)


def _tc_stage2(acc2, z2c, cdim, blk):
    n = z2c.shape[0]

    def body(acc_ref, z2c_ref, out_ref):
        agg = acc_ref[0] + acc_ref[1]
        agg_sum = agg[:, :cdim]
        z2cb = z2c_ref[...]
        z2 = z2cb[:, :cdim]
        cnt = z2cb[:, cdim:cdim + 1]
        pre = agg_sum / cnt + z2
        nrm = jnp.sqrt(jnp.sum(pre * pre, axis=1, keepdims=True))
        out_ref[...] = pre / jnp.maximum(nrm, 1e-12)

    grid = (n // blk,)
    return pl.pallas_call(
        body,
        grid=grid,
        in_specs=[
            pl.BlockSpec((_NC, blk, 64), lambda i: (0, i, 0)),
            pl.BlockSpec((blk, 128), lambda i: (i, 0)),
        ],
        out_specs=pl.BlockSpec((blk, cdim), lambda i: (i, 0)),
        out_shape=jax.ShapeDtypeStruct((n, cdim), jnp.float32),
    )(acc2, z2c)


def kernel(x, Wl1, bl1, Wr1, W1, b1, Wl2, bl2, Wr2, edge_index):
    n, d = x.shape
    e = edge_index.shape[1]
    cdim = Wl2.shape[0]

    k_chunk = 112
    e1 = _NS * 180 * k_chunk
    e2 = _NS * _NC * 92 * k_chunk
    n_acc = n + 240

    def pad_edges(e_pad):
        src = jnp.concatenate(
            [edge_index[0], jnp.zeros((e_pad - e,), jnp.int32)]
        ).reshape(-1, k_chunk)
        dst = jnp.concatenate(
            [edge_index[1],
             n + (jnp.arange(e_pad - e, dtype=jnp.int32) % (n_acc - n))]
        ).reshape(-1, k_chunk)
        return src, dst

    src2d, dst2d = pad_edges(e1)
    src2db, dst2db = pad_edges(e2)

    acc1, cnt1 = _sc_segment_sum(x, src2d, dst2d, n_acc, k_chunk, cpp=36,
                                 with_counts=True, col_split=True)

    bl1r = bl1.reshape(1, -1)
    b1r = b1.reshape(1, -1)
    bl2r = bl2.reshape(1, -1)
    cnt1t = cnt1.T
    y2p, z2c = _tc_stage1(x, acc1, cnt1t, Wl1, bl1r, Wr1, W1, b1r, Wl2, Wr2,
                          bl2r, blk=2000)

    acc2, = _sc_segment_sum(y2p, src2db, dst2db, n_acc, k_chunk, cpp=92,
                            with_counts=False, col_split=False)

    return _tc_stage2(acc2, z2c, cdim, blk=2000)
